# baseline (speedup 1.0000x reference)
import functools

import jax
import jax.numpy as jnp
from jax import lax
from jax.experimental import pallas as pl
from jax.experimental.pallas import tpu as pltpu
from jax.experimental.pallas import tpu_sc as plsc

N_TOK = 65536
K_CODES = 512
DIM = 32
BLK = 2048
CHUNK = 128


def _dist_argmin_body(z_ref, emb_ref, idx_ref, loss_ref):
    i = pl.program_id(0)
    z = z_ref[...]
    emb = emb_ref[...]
    cross_t = lax.dot_general(emb, z, (((1,), (1,)), ((), ())),
                              preferred_element_type=jnp.float32)
    e_sq = jnp.sum(emb * emb, axis=1, keepdims=True)
    dT = e_sq - 2.0 * cross_t
    idx_ref[...] = jnp.argmin(dT, axis=0).astype(jnp.int32)
    blk_loss = jnp.sum(jnp.min(dT, axis=0)) + jnp.sum(z * z)

    @pl.when(i == 0)
    def _init():
        loss_ref[...] = jnp.zeros((1, 1), jnp.float32)

    loss_ref[...] = loss_ref[...] + blk_loss


def _dist_argmin(z_e, embeddings):
    grid = N_TOK // BLK
    return pl.pallas_call(
        _dist_argmin_body,
        grid=(grid,),
        in_specs=[
            pl.BlockSpec((BLK, DIM), lambda i: (i, 0)),
            pl.BlockSpec((K_CODES, DIM), lambda i: (0, 0)),
        ],
        out_specs=[
            pl.BlockSpec((BLK,), lambda i: (i,)),
            pl.BlockSpec((1, 1), lambda i: (0, 0)),
        ],
        out_shape=[
            jax.ShapeDtypeStruct((N_TOK,), jnp.int32),
            jax.ShapeDtypeStruct((1, 1), jnp.float32),
        ],
    )(z_e, embeddings)


@functools.cache
def _make_sc_gather():
    info = plsc.get_sparse_core_info()
    nc, ns = info.num_cores, info.num_subcores
    nw = nc * ns
    tok_per_w = N_TOK // nw
    n_chunks = tok_per_w // CHUNK
    mesh = plsc.VectorSubcoreMesh(core_axis_name="c", subcore_axis_name="s")

    @functools.partial(
        pl.kernel,
        mesh=mesh,
        out_type=jax.ShapeDtypeStruct((N_TOK, DIM), jnp.float32),
        scratch_types=[
            pltpu.VMEM((tok_per_w,), jnp.int32),
            pltpu.VMEM((tok_per_w, DIM), jnp.float32),
            pltpu.SemaphoreType.DMA,
        ],
        compiler_params=pltpu.CompilerParams(use_tc_tiling_on_sc=False),
    )
    def gather(table_hbm, idx_hbm, out_hbm, idx_v, rows_v, sem):
        wid = lax.axis_index("s") * nc + lax.axis_index("c")
        base = wid * tok_per_w
        pltpu.sync_copy(idx_hbm.at[pl.ds(base, tok_per_w)], idx_v)
        copies = [
            pltpu.async_copy(
                table_hbm.at[idx_v.at[pl.ds(j * CHUNK, CHUNK)]],
                rows_v.at[pl.ds(j * CHUNK, CHUNK)],
                sem,
            )
            for j in range(n_chunks)
        ]
        for c in copies:
            c.wait()
        pltpu.sync_copy(rows_v, out_hbm.at[pl.ds(base, tok_per_w)])

    return gather


def kernel(z_e, embeddings):
    indices, loss_sum = _dist_argmin(z_e, embeddings)
    z_q_st = _make_sc_gather()(embeddings, indices)
    vq_loss = (1.25 / (N_TOK * DIM)) * loss_sum.reshape(())
    return (z_q_st, vq_loss, indices)


# speedup vs baseline: 1.2028x; 1.2028x over previous
import functools

import jax
import jax.numpy as jnp
from jax import lax
from jax.experimental import pallas as pl
from jax.experimental.pallas import tpu as pltpu
from jax.experimental.pallas import tpu_sc as plsc

N_TOK = 65536
K_CODES = 512
DIM = 32
BLK = 2048
CHUNK = 128


def _dist_argmin_body(z_ref, emb_ref, idx_ref, loss_ref):
    i = pl.program_id(0)
    z = z_ref[...]
    emb = emb_ref[...]
    cross_t = lax.dot_general(emb, z, (((1,), (0,)), ((), ())),
                              preferred_element_type=jnp.float32)
    e_sq = jnp.sum(emb * emb, axis=1, keepdims=True)
    dT = e_sq - 2.0 * cross_t
    idx_ref[...] = jnp.argmin(dT, axis=0).astype(jnp.int32)
    blk_loss = jnp.sum(jnp.min(dT, axis=0)) + jnp.sum(z * z)

    @pl.when(i == 0)
    def _init():
        loss_ref[...] = jnp.zeros((1, 1), jnp.float32)

    loss_ref[...] = loss_ref[...] + blk_loss


def _dist_argmin(z_t, embeddings):
    grid = N_TOK // BLK
    return pl.pallas_call(
        _dist_argmin_body,
        grid=(grid,),
        in_specs=[
            pl.BlockSpec((DIM, BLK), lambda i: (0, i)),
            pl.BlockSpec((K_CODES, DIM), lambda i: (0, 0)),
        ],
        out_specs=[
            pl.BlockSpec((BLK,), lambda i: (i,)),
            pl.BlockSpec((1, 1), lambda i: (0, 0)),
        ],
        out_shape=[
            jax.ShapeDtypeStruct((N_TOK,), jnp.int32),
            jax.ShapeDtypeStruct((1, 1), jnp.float32),
        ],
    )(z_t, embeddings)


@functools.cache
def _make_sc_gather():
    info = plsc.get_sparse_core_info()
    nc, ns = info.num_cores, info.num_subcores
    nw = nc * ns
    tok_per_w = N_TOK // nw
    n_chunks = tok_per_w // CHUNK
    mesh = plsc.VectorSubcoreMesh(core_axis_name="c", subcore_axis_name="s")

    @functools.partial(
        pl.kernel,
        mesh=mesh,
        out_type=jax.ShapeDtypeStruct((N_TOK, DIM), jnp.float32),
        scratch_types=[
            pltpu.VMEM((tok_per_w,), jnp.int32),
            pltpu.VMEM((tok_per_w, DIM), jnp.float32),
            pltpu.SemaphoreType.DMA,
        ],
        compiler_params=pltpu.CompilerParams(use_tc_tiling_on_sc=False),
    )
    def gather(table_hbm, idx_hbm, out_hbm, idx_v, rows_v, sem):
        wid = lax.axis_index("s") * nc + lax.axis_index("c")
        base = wid * tok_per_w
        pltpu.sync_copy(idx_hbm.at[pl.ds(base, tok_per_w)], idx_v)
        copies = [
            pltpu.async_copy(
                table_hbm.at[idx_v.at[pl.ds(j * CHUNK, CHUNK)]],
                rows_v.at[pl.ds(j * CHUNK, CHUNK)],
                sem,
            )
            for j in range(n_chunks)
        ]
        for c in copies:
            c.wait()
        pltpu.sync_copy(rows_v, out_hbm.at[pl.ds(base, tok_per_w)])

    return gather


def kernel(z_e, embeddings):
    indices, loss_sum = _dist_argmin(z_e.T, embeddings)
    z_q_st = _make_sc_gather()(embeddings, indices)
    vq_loss = (1.25 / (N_TOK * DIM)) * loss_sum.reshape(())
    return (z_q_st, vq_loss, indices)


# speedup vs baseline: 1.7739x; 1.4748x over previous
import functools

import jax
import jax.numpy as jnp
from jax import lax
from jax.experimental import pallas as pl
from jax.experimental.pallas import tpu as pltpu
from jax.experimental.pallas import tpu_sc as plsc

N_TOK = 65536
K_CODES = 512
DIM = 32
BLK = 2048
CHUNK = 128


def _dist_argmin_body(z_ref, emb_ref, idx_ref, loss_ref):
    i = pl.program_id(0)
    z = z_ref[...]
    emb = emb_ref[...]
    cross_t = lax.dot_general(emb, z, (((1,), (0,)), ((), ())),
                              preferred_element_type=jnp.float32)
    e_sq = jnp.sum(emb * emb, axis=1, keepdims=True)
    dT = e_sq - 2.0 * cross_t
    idx_ref[...] = jnp.argmin(dT, axis=0).astype(jnp.int32)
    blk_loss = jnp.sum(jnp.min(dT, axis=0)) + jnp.sum(z * z)

    @pl.when(i == 0)
    def _init():
        loss_ref[...] = jnp.zeros((1, 1), jnp.float32)

    loss_ref[...] = loss_ref[...] + blk_loss


def _dist_argmin(z_t, embeddings):
    grid = N_TOK // BLK
    return pl.pallas_call(
        _dist_argmin_body,
        grid=(grid,),
        in_specs=[
            pl.BlockSpec((DIM, BLK), lambda i: (0, i)),
            pl.BlockSpec((K_CODES, DIM), lambda i: (0, 0)),
        ],
        out_specs=[
            pl.BlockSpec((BLK,), lambda i: (i,)),
            pl.BlockSpec((1, 1), lambda i: (0, 0)),
        ],
        out_shape=[
            jax.ShapeDtypeStruct((N_TOK,), jnp.int32),
            jax.ShapeDtypeStruct((1, 1), jnp.float32),
        ],
    )(z_t, embeddings)


@functools.cache
def _make_sc_gather():
    info = plsc.get_sparse_core_info()
    nc, ns = info.num_cores, info.num_subcores
    nw = nc * ns
    tok_per_w = N_TOK // nw
    lanes = info.num_lanes
    n_batches = tok_per_w // lanes
    mesh = plsc.VectorSubcoreMesh(core_axis_name="c", subcore_axis_name="s")

    @functools.partial(
        pl.kernel,
        mesh=mesh,
        out_type=jax.ShapeDtypeStruct((DIM, N_TOK), jnp.float32),
        scratch_types=[
            pltpu.VMEM((tok_per_w,), jnp.int32),
            pltpu.VMEM((DIM, K_CODES), jnp.float32),
            pltpu.VMEM((DIM, tok_per_w), jnp.float32),
        ],
        compiler_params=pltpu.CompilerParams(use_tc_tiling_on_sc=True,
                                             needs_layout_passes=False),
    )
    def gather(table_hbm, idx_hbm, out_hbm, idx_v, table_v, zq_v):
        wid = lax.axis_index("s") * nc + lax.axis_index("c")
        base = wid * tok_per_w
        pltpu.sync_copy(table_hbm, table_v)
        pltpu.sync_copy(idx_hbm.at[pl.ds(base, tok_per_w)], idx_v)
        lane_iota = lax.iota(jnp.int32, lanes)

        def body(b, _):
            idx16 = idx_v[pl.ds(b * lanes, lanes)]
            tok16 = b * lanes + lane_iota
            for d in range(DIM):
                d16 = jnp.full((lanes,), d, jnp.int32)
                vals = plsc.load_gather(table_v, [d16, idx16])
                plsc.store_scatter(zq_v, [d16, tok16], vals)
            return _

        lax.fori_loop(0, n_batches, body, None)
        pltpu.sync_copy(zq_v, out_hbm.at[:, pl.ds(base, tok_per_w)])

    return gather


def kernel(z_e, embeddings):
    indices, loss_sum = _dist_argmin(z_e.T, embeddings)
    z_q_t = _make_sc_gather()(embeddings.T, indices)
    z_q_st = z_q_t.T
    vq_loss = (1.25 / (N_TOK * DIM)) * loss_sum.reshape(())
    return (z_q_st, vq_loss, indices)


# speedup vs baseline: 1.9336x; 1.0900x over previous
import functools

import jax
import jax.numpy as jnp
from jax import lax
from jax.experimental import pallas as pl
from jax.experimental.pallas import tpu as pltpu
from jax.experimental.pallas import tpu_sc as plsc

N_TOK = 65536
K_CODES = 512
DIM = 32
BLK = 4096
CHUNK = 128


def _dist_argmin_body(z_ref, emb_ref, idx_ref, loss_ref):
    i = pl.program_id(0)
    z = z_ref[...]
    emb = emb_ref[...]
    cross_m2 = lax.dot_general(-2.0 * emb, z, (((1,), (0,)), ((), ())),
                               preferred_element_type=jnp.float32)
    e_sq = jnp.sum(emb * emb, axis=1, keepdims=True)
    dT = e_sq + cross_m2
    idx_ref[...] = jnp.argmin(dT, axis=0).astype(jnp.int32)
    blk_loss = jnp.sum(jnp.min(dT, axis=0)) + jnp.sum(z * z)

    @pl.when(i == 0)
    def _init():
        loss_ref[...] = jnp.zeros((1, 1), jnp.float32)

    loss_ref[...] = loss_ref[...] + blk_loss


def _dist_argmin(z_t, embeddings):
    grid = N_TOK // BLK
    return pl.pallas_call(
        _dist_argmin_body,
        grid=(grid,),
        in_specs=[
            pl.BlockSpec((DIM, BLK), lambda i: (0, i)),
            pl.BlockSpec((K_CODES, DIM), lambda i: (0, 0)),
        ],
        out_specs=[
            pl.BlockSpec((BLK,), lambda i: (i,)),
            pl.BlockSpec((1, 1), lambda i: (0, 0)),
        ],
        out_shape=[
            jax.ShapeDtypeStruct((N_TOK,), jnp.int32),
            jax.ShapeDtypeStruct((1, 1), jnp.float32),
        ],
    )(z_t, embeddings)


@functools.cache
def _make_sc_gather():
    info = plsc.get_sparse_core_info()
    nc, ns = info.num_cores, info.num_subcores
    nw = nc * ns
    tok_per_w = N_TOK // nw
    lanes = info.num_lanes
    n_batches = tok_per_w // lanes
    mesh = plsc.VectorSubcoreMesh(core_axis_name="c", subcore_axis_name="s")

    @functools.partial(
        pl.kernel,
        mesh=mesh,
        out_type=jax.ShapeDtypeStruct((DIM, N_TOK), jnp.float32),
        scratch_types=[
            pltpu.VMEM((tok_per_w,), jnp.int32),
            pltpu.VMEM((DIM, K_CODES), jnp.float32),
            pltpu.VMEM((DIM, tok_per_w), jnp.float32),
        ],
        compiler_params=pltpu.CompilerParams(use_tc_tiling_on_sc=True,
                                             needs_layout_passes=False),
    )
    def gather(table_hbm, idx_hbm, out_hbm, idx_v, table_v, zq_v):
        wid = lax.axis_index("s") * nc + lax.axis_index("c")
        base = wid * tok_per_w
        pltpu.sync_copy(table_hbm, table_v)
        pltpu.sync_copy(idx_hbm.at[pl.ds(base, tok_per_w)], idx_v)
        lane_iota = lax.iota(jnp.int32, lanes)

        def body(b, _):
            for u in range(2):
                bb = b * 2 + u
                idx16 = idx_v[pl.ds(bb * lanes, lanes)]
                tok16 = bb * lanes + lane_iota
                for d in range(DIM):
                    d16 = jnp.full((lanes,), d, jnp.int32)
                    vals = plsc.load_gather(table_v, [d16, idx16])
                    plsc.store_scatter(zq_v, [d16, tok16], vals)
            return _

        lax.fori_loop(0, n_batches // 2, body, None)
        pltpu.sync_copy(zq_v, out_hbm.at[:, pl.ds(base, tok_per_w)])

    return gather


def kernel(z_e, embeddings):
    indices, loss_sum = _dist_argmin(z_e.T, embeddings)
    z_q_t = _make_sc_gather()(embeddings.T, indices)
    z_q_st = z_q_t.T
    vq_loss = (1.25 / (N_TOK * DIM)) * loss_sum.reshape(())
    return (z_q_st, vq_loss, indices)


# speedup vs baseline: 2.0876x; 1.0797x over previous
import functools

import jax
import jax.numpy as jnp
from jax import lax
from jax.experimental import pallas as pl
from jax.experimental.pallas import tpu as pltpu
from jax.experimental.pallas import tpu_sc as plsc

N_TOK = 65536
K_CODES = 512
DIM = 32
BLK = 4096
CHUNK = 128


def _dist_argmin_body(z_ref, emb_ref, idx_ref, loss_ref):
    i = pl.program_id(0)
    z = z_ref[...]
    emb = emb_ref[...]
    cross_m2 = lax.dot_general(-2.0 * emb, z, (((1,), (0,)), ((), ())),
                               preferred_element_type=jnp.float32)
    e_sq = jnp.sum(emb * emb, axis=1, keepdims=True)
    dT = e_sq + cross_m2
    idx_ref[...] = jnp.argmin(dT, axis=0).astype(jnp.int32)
    blk_loss = jnp.sum(jnp.min(dT, axis=0)) + jnp.sum(z * z)

    @pl.when(i == 0)
    def _init():
        loss_ref[...] = jnp.zeros((1, 1), jnp.float32)

    loss_ref[...] = loss_ref[...] + blk_loss


def _dist_argmin(z_t, embeddings):
    grid = N_TOK // BLK
    return pl.pallas_call(
        _dist_argmin_body,
        grid=(grid,),
        in_specs=[
            pl.BlockSpec((DIM, BLK), lambda i: (0, i)),
            pl.BlockSpec((K_CODES, DIM), lambda i: (0, 0)),
        ],
        out_specs=[
            pl.BlockSpec((BLK,), lambda i: (i,)),
            pl.BlockSpec((1, 1), lambda i: (0, 0)),
        ],
        out_shape=[
            jax.ShapeDtypeStruct((N_TOK,), jnp.int32),
            jax.ShapeDtypeStruct((1, 1), jnp.float32),
        ],
    )(z_t, embeddings)


@functools.cache
def _make_sc_gather():
    info = plsc.get_sparse_core_info()
    nc, ns = info.num_cores, info.num_subcores
    nw = nc * ns
    tok_per_w = N_TOK // nw
    lanes = info.num_lanes
    n_batches = tok_per_w // lanes
    mesh = plsc.VectorSubcoreMesh(core_axis_name="c", subcore_axis_name="s")

    @functools.partial(
        pl.kernel,
        mesh=mesh,
        out_type=jax.ShapeDtypeStruct((DIM, N_TOK), jnp.float32),
        scratch_types=[
            pltpu.VMEM((tok_per_w,), jnp.int32),
            pltpu.VMEM((DIM, K_CODES), jnp.float32),
            pltpu.VMEM((DIM, tok_per_w), jnp.float32),
        ],
        compiler_params=pltpu.CompilerParams(use_tc_tiling_on_sc=True,
                                             needs_layout_passes=False),
    )
    def gather(table_hbm, idx_hbm, out_hbm, idx_v, table_v, zq_v):
        wid = lax.axis_index("s") * nc + lax.axis_index("c")
        base = wid * tok_per_w
        pltpu.sync_copy(table_hbm, table_v)
        pltpu.sync_copy(idx_hbm.at[pl.ds(base, tok_per_w)], idx_v)
        lane_iota = lax.iota(jnp.int32, lanes)

        @plsc.parallel_loop(0, n_batches, step=1, unroll=4)
        def _batches(b):
            idx16 = idx_v[pl.ds(b * lanes, lanes)]
            tok16 = b * lanes + lane_iota
            for d in range(DIM):
                d16 = jnp.full((lanes,), d, jnp.int32)
                vals = plsc.load_gather(table_v, [d16, idx16])
                plsc.store_scatter(zq_v, [d16, tok16], vals)
        pltpu.sync_copy(zq_v, out_hbm.at[:, pl.ds(base, tok_per_w)])

    return gather


def kernel(z_e, embeddings):
    indices, loss_sum = _dist_argmin(z_e.T, embeddings)
    z_q_t = _make_sc_gather()(embeddings.T, indices)
    z_q_st = z_q_t.T
    vq_loss = (1.25 / (N_TOK * DIM)) * loss_sum.reshape(())
    return (z_q_st, vq_loss, indices)


# speedup vs baseline: 2.3165x; 1.1097x over previous
import functools

import jax
import jax.numpy as jnp
from jax import lax
from jax.experimental import pallas as pl
from jax.experimental.pallas import tpu as pltpu
from jax.experimental.pallas import tpu_sc as plsc

N_TOK = 65536
K_CODES = 512
DIM = 32
BLK = 8192
CHUNK = 128


def _dist_argmin_body(z_ref, emb_ref, idx_ref, loss_ref):
    i = pl.program_id(0)
    z = z_ref[...]
    emb = emb_ref[...]
    cross_m2 = lax.dot_general(-2.0 * emb, z, (((1,), (0,)), ((), ())),
                               preferred_element_type=jnp.float32)
    e_sq = jnp.sum(emb * emb, axis=1, keepdims=True)
    dT = e_sq + cross_m2
    idx_ref[...] = jnp.argmin(dT, axis=0).astype(jnp.int32)
    blk_loss = jnp.sum(jnp.min(dT, axis=0)) + jnp.sum(z * z)

    @pl.when(i == 0)
    def _init():
        loss_ref[...] = jnp.zeros((1, 1), jnp.float32)

    loss_ref[...] = loss_ref[...] + blk_loss


def _dist_argmin(z_t, embeddings):
    grid = N_TOK // BLK
    return pl.pallas_call(
        _dist_argmin_body,
        grid=(grid,),
        in_specs=[
            pl.BlockSpec((DIM, BLK), lambda i: (0, i)),
            pl.BlockSpec((K_CODES, DIM), lambda i: (0, 0)),
        ],
        out_specs=[
            pl.BlockSpec((BLK,), lambda i: (i,)),
            pl.BlockSpec((1, 1), lambda i: (0, 0)),
        ],
        out_shape=[
            jax.ShapeDtypeStruct((N_TOK,), jnp.int32),
            jax.ShapeDtypeStruct((1, 1), jnp.float32),
        ],
    )(z_t, embeddings)


@functools.cache
def _make_sc_gather():
    info = plsc.get_sparse_core_info()
    nc, ns = info.num_cores, info.num_subcores
    nw = nc * ns
    tok_per_w = N_TOK // nw
    lanes = info.num_lanes
    n_batches = tok_per_w // lanes
    mesh = plsc.VectorSubcoreMesh(core_axis_name="c", subcore_axis_name="s")

    @functools.partial(
        pl.kernel,
        mesh=mesh,
        out_type=jax.ShapeDtypeStruct((DIM, N_TOK), jnp.float32),
        scratch_types=[
            pltpu.VMEM((tok_per_w,), jnp.int32),
            pltpu.VMEM((DIM, K_CODES), jnp.float32),
            pltpu.VMEM((DIM, tok_per_w), jnp.float32),
        ],
        compiler_params=pltpu.CompilerParams(use_tc_tiling_on_sc=True,
                                             needs_layout_passes=False),
    )
    def gather(table_hbm, idx_hbm, out_hbm, idx_v, table_v, zq_v):
        wid = lax.axis_index("s") * nc + lax.axis_index("c")
        base = wid * tok_per_w
        pltpu.sync_copy(table_hbm, table_v)
        pltpu.sync_copy(idx_hbm.at[pl.ds(base, tok_per_w)], idx_v)
        lane_iota = lax.iota(jnp.int32, lanes)

        @plsc.parallel_loop(0, n_batches, step=1, unroll=8)
        def _batches(b):
            idx16 = idx_v[pl.ds(b * lanes, lanes)]
            tok16 = b * lanes + lane_iota
            for d in range(DIM):
                d16 = jnp.full((lanes,), d, jnp.int32)
                vals = plsc.load_gather(table_v, [d16, idx16])
                plsc.store_scatter(zq_v, [d16, tok16], vals)
        pltpu.sync_copy(zq_v, out_hbm.at[:, pl.ds(base, tok_per_w)])

    return gather


def kernel(z_e, embeddings):
    indices, loss_sum = _dist_argmin(z_e.T, embeddings)
    z_q_t = _make_sc_gather()(embeddings.T, indices)
    z_q_st = z_q_t.T
    vq_loss = (1.25 / (N_TOK * DIM)) * loss_sum.reshape(())
    return (z_q_st, vq_loss, indices)


# speedup vs baseline: 2.3365x; 1.0086x over previous
import functools

import jax
import jax.numpy as jnp
from jax import lax
from jax.experimental import pallas as pl
from jax.experimental.pallas import tpu as pltpu
from jax.experimental.pallas import tpu_sc as plsc

N_TOK = 65536
K_CODES = 512
DIM = 32
BLK = 16384
CHUNK = 128


def _dist_argmin_body(z_ref, emb_ref, idx_ref, loss_ref):
    i = pl.program_id(0)
    z = z_ref[...]
    emb = emb_ref[...]
    cross_m2 = lax.dot_general(-2.0 * emb, z, (((1,), (0,)), ((), ())),
                               preferred_element_type=jnp.float32)
    e_sq = jnp.sum(emb * emb, axis=1, keepdims=True)
    dT = e_sq + cross_m2
    idx_ref[...] = jnp.argmin(dT, axis=0).astype(jnp.int32)
    blk_loss = jnp.sum(jnp.min(dT, axis=0)) + jnp.sum(z * z)

    @pl.when(i == 0)
    def _init():
        loss_ref[...] = jnp.zeros((1, 1), jnp.float32)

    loss_ref[...] = loss_ref[...] + blk_loss


def _dist_argmin(z_t, embeddings):
    grid = N_TOK // BLK
    return pl.pallas_call(
        _dist_argmin_body,
        grid=(grid,),
        in_specs=[
            pl.BlockSpec((DIM, BLK), lambda i: (0, i)),
            pl.BlockSpec((K_CODES, DIM), lambda i: (0, 0)),
        ],
        out_specs=[
            pl.BlockSpec((BLK,), lambda i: (i,)),
            pl.BlockSpec((1, 1), lambda i: (0, 0)),
        ],
        out_shape=[
            jax.ShapeDtypeStruct((N_TOK,), jnp.int32),
            jax.ShapeDtypeStruct((1, 1), jnp.float32),
        ],
    )(z_t, embeddings)


@functools.cache
def _make_sc_gather():
    info = plsc.get_sparse_core_info()
    nc, ns = info.num_cores, info.num_subcores
    nw = nc * ns
    tok_per_w = N_TOK // nw
    lanes = info.num_lanes
    n_batches = tok_per_w // lanes
    mesh = plsc.VectorSubcoreMesh(core_axis_name="c", subcore_axis_name="s")

    @functools.partial(
        pl.kernel,
        mesh=mesh,
        out_type=jax.ShapeDtypeStruct((DIM, N_TOK), jnp.float32),
        scratch_types=[
            pltpu.VMEM((tok_per_w,), jnp.int32),
            pltpu.VMEM((DIM, K_CODES), jnp.float32),
            pltpu.VMEM((DIM, tok_per_w), jnp.float32),
        ],
        compiler_params=pltpu.CompilerParams(use_tc_tiling_on_sc=True,
                                             needs_layout_passes=False),
    )
    def gather(table_hbm, idx_hbm, out_hbm, idx_v, table_v, zq_v):
        wid = lax.axis_index("s") * nc + lax.axis_index("c")
        base = wid * tok_per_w
        pltpu.sync_copy(table_hbm, table_v)
        pltpu.sync_copy(idx_hbm.at[pl.ds(base, tok_per_w)], idx_v)
        lane_iota = lax.iota(jnp.int32, lanes)

        @plsc.parallel_loop(0, n_batches, step=1, unroll=8)
        def _batches(b):
            idx16 = idx_v[pl.ds(b * lanes, lanes)]
            tok16 = b * lanes + lane_iota
            for d in range(DIM):
                d16 = jnp.full((lanes,), d, jnp.int32)
                vals = plsc.load_gather(table_v, [d16, idx16])
                plsc.store_scatter(zq_v, [d16, tok16], vals)
        pltpu.sync_copy(zq_v, out_hbm.at[:, pl.ds(base, tok_per_w)])

    return gather


def kernel(z_e, embeddings):
    indices, loss_sum = _dist_argmin(z_e.T, embeddings)
    z_q_t = _make_sc_gather()(embeddings.T, indices)
    z_q_st = z_q_t.T
    vq_loss = (1.25 / (N_TOK * DIM)) * loss_sum.reshape(())
    return (z_q_st, vq_loss, indices)


# speedup vs baseline: 2.3860x; 1.0212x over previous
import functools

import jax
import jax.numpy as jnp
from jax import lax
from jax.experimental import pallas as pl
from jax.experimental.pallas import tpu as pltpu
from jax.experimental.pallas import tpu_sc as plsc

N_TOK = 65536
K_CODES = 512
DIM = 32
BLK = 16384
CHUNK = 128


def _dist_argmin_body(z_ref, emb_ref, idx_ref, loss_ref):
    i = pl.program_id(0)
    z = z_ref[...]
    emb_t = emb_ref[...]
    cross_m2 = lax.dot_general(-2.0 * emb_t, z, (((0,), (0,)), ((), ())),
                               preferred_element_type=jnp.float32)
    e_sq = jnp.sum(emb_t * emb_t, axis=0)[:, None]
    dT = e_sq + cross_m2
    idx_ref[...] = jnp.argmin(dT, axis=0).astype(jnp.int32)
    blk_loss = jnp.sum(jnp.min(dT, axis=0)) + jnp.sum(z * z)

    @pl.when(i == 0)
    def _init():
        loss_ref[...] = jnp.zeros((1, 1), jnp.float32)

    loss_ref[...] = loss_ref[...] + blk_loss


def _dist_argmin(z_t, emb_t):
    grid = N_TOK // BLK
    return pl.pallas_call(
        _dist_argmin_body,
        grid=(grid,),
        in_specs=[
            pl.BlockSpec((DIM, BLK), lambda i: (0, i)),
            pl.BlockSpec((DIM, K_CODES), lambda i: (0, 0)),
        ],
        out_specs=[
            pl.BlockSpec((BLK,), lambda i: (i,)),
            pl.BlockSpec((1, 1), lambda i: (0, 0)),
        ],
        out_shape=[
            jax.ShapeDtypeStruct((N_TOK,), jnp.int32),
            jax.ShapeDtypeStruct((1, 1), jnp.float32),
        ],
    )(z_t, emb_t)


@functools.cache
def _make_sc_gather():
    info = plsc.get_sparse_core_info()
    nc, ns = info.num_cores, info.num_subcores
    nw = nc * ns
    tok_per_w = N_TOK // nw
    lanes = info.num_lanes
    n_batches = tok_per_w // lanes
    mesh = plsc.VectorSubcoreMesh(core_axis_name="c", subcore_axis_name="s")

    @functools.partial(
        pl.kernel,
        mesh=mesh,
        out_type=jax.ShapeDtypeStruct((DIM, N_TOK), jnp.float32),
        scratch_types=[
            pltpu.VMEM((tok_per_w,), jnp.int32),
            pltpu.VMEM((DIM, K_CODES), jnp.float32),
            pltpu.VMEM((DIM, tok_per_w), jnp.float32),
        ],
        compiler_params=pltpu.CompilerParams(use_tc_tiling_on_sc=True,
                                             needs_layout_passes=False),
    )
    def gather(table_hbm, idx_hbm, out_hbm, idx_v, table_v, zq_v):
        wid = lax.axis_index("s") * nc + lax.axis_index("c")
        base = wid * tok_per_w
        pltpu.sync_copy(table_hbm, table_v)
        pltpu.sync_copy(idx_hbm.at[pl.ds(base, tok_per_w)], idx_v)
        lane_iota = lax.iota(jnp.int32, lanes)

        @plsc.parallel_loop(0, n_batches, step=1, unroll=8)
        def _batches(b):
            idx16 = idx_v[pl.ds(b * lanes, lanes)]
            tok16 = b * lanes + lane_iota
            for d in range(DIM):
                d16 = jnp.full((lanes,), d, jnp.int32)
                vals = plsc.load_gather(table_v, [d16, idx16])
                plsc.store_scatter(zq_v, [d16, tok16], vals)
        pltpu.sync_copy(zq_v, out_hbm.at[:, pl.ds(base, tok_per_w)])

    return gather


def kernel(z_e, embeddings):
    indices, loss_sum = _dist_argmin(z_e.T, embeddings.T)
    z_q_t = _make_sc_gather()(embeddings.T, indices)
    z_q_st = z_q_t.T
    vq_loss = (1.25 / (N_TOK * DIM)) * loss_sum.reshape(())
    return (z_q_st, vq_loss, indices)
